# Initial kernel scaffold; baseline (speedup 1.0000x reference)
import jax
import jax.numpy as jnp
from jax.experimental import pallas as pl


def kernel(x, W_up, W_down, gate_W, gate_b):
    raise NotImplementedError("write your pallas kernel here")


# speedup vs baseline: 1.2308x; 1.2308x over previous
import functools

import jax
import jax.numpy as jnp
from jax import lax
from jax.experimental import pallas as pl
from jax.experimental.pallas import tpu as pltpu
from jax.experimental.pallas import tpu_sc as plsc

D_MODEL = 768
D_FF = 1024
N_EXP = 64
CAP = 160
N_TOK = 2048
N_SLOT = N_EXP * CAP

NW = 32
SLOTS_PER_W = N_SLOT // NW
GCHUNK = 80
TOKS_PER_W = N_TOK // NW


def _gate_body(x_ref, gw_ref, gb_ref, slot_ref, cnt_ref, tok_ref):
    x = x_ref[...]
    gw = gw_ref[...]
    logits = lax.dot_general(x, gw, (((1,), (1,)), ((), ())),
                             preferred_element_type=jnp.float32)
    logits = logits + gb_ref[...]
    m = jnp.max(logits, axis=1, keepdims=True)
    lane = lax.broadcasted_iota(jnp.int32, (N_TOK, N_EXP), 1)
    eid = jnp.min(jnp.where(logits == m, lane, N_EXP), axis=1, keepdims=True)
    onehot = (lane == eid)
    r_i = lax.broadcasted_iota(jnp.int32, (N_TOK, N_TOK), 0)
    c_i = lax.broadcasted_iota(jnp.int32, (N_TOK, N_TOK), 1)
    tri = (c_i < r_i).astype(jnp.bfloat16)
    oh_b = onehot.astype(jnp.bfloat16)
    csum = lax.dot_general(tri, oh_b, (((1,), (0,)), ((), ())),
                           preferred_element_type=jnp.float32)
    rank = jnp.sum(jnp.where(onehot, csum, 0.0), axis=1, keepdims=True)
    rank = rank.astype(jnp.int32)
    counts = jnp.sum(oh_b.astype(jnp.float32), axis=0, keepdims=True)
    cnt_ref[...] = counts.astype(jnp.int32)
    cmin = jnp.min(counts, axis=1, keepdims=True)
    elane = lax.broadcasted_iota(jnp.int32, (1, N_EXP), 1)
    emin = jnp.min(jnp.where(counts == cmin, elane, N_EXP), axis=1,
                   keepdims=True)
    trash = emin * CAP + (CAP - 1)
    kept = rank < CAP
    slot = jnp.where(kept, eid * CAP + rank, trash)
    slot_ref[...] = slot
    tvec = lax.broadcasted_iota(jnp.int32, (N_TOK, 1), 0).astype(jnp.float32)
    ot = onehot.astype(jnp.float32) * tvec
    rmat = (lax.broadcasted_iota(jnp.int32, (N_TOK, CAP), 1) == rank)
    tok = lax.dot_general(ot, rmat.astype(jnp.float32),
                          (((0,), (0,)), ((), ())),
                          preferred_element_type=jnp.float32)
    tok_ref[...] = tok.astype(jnp.int32)


def _gate_call(xf, gate_W, gate_b):
    return pl.pallas_call(
        _gate_body,
        out_shape=[
            jax.ShapeDtypeStruct((N_TOK, 1), jnp.int32),
            jax.ShapeDtypeStruct((1, N_EXP), jnp.int32),
            jax.ShapeDtypeStruct((N_EXP, CAP), jnp.int32),
        ],
    )(xf, gate_W, gate_b.reshape(1, N_EXP))


def _dispatch_body(tok_hbm, x_hbm, xg_hbm, idx_v, rows_v, gsem, wsem):
    wid = lax.axis_index("s") * 2 + lax.axis_index("c")
    base = wid * SLOTS_PER_W
    pltpu.sync_copy(tok_hbm.at[pl.ds(base, SLOTS_PER_W)], idx_v)
    nk = SLOTS_PER_W // GCHUNK
    gcp = [None, None]
    wcp = [None, None]
    for k in range(nk):
        b = k % 2
        if wcp[b] is not None:
            wcp[b].wait()
        gcp[b] = pltpu.async_copy(
            x_hbm.at[idx_v.at[pl.ds(k * GCHUNK, GCHUNK)]], rows_v.at[b],
            gsem.at[b])
        if k >= 1:
            pb = (k - 1) % 2
            gcp[pb].wait()
            wcp[pb] = pltpu.async_copy(
                rows_v.at[pb], xg_hbm.at[pl.ds(base + (k - 1) * GCHUNK, GCHUNK)],
                wsem.at[pb])
    lb = (nk - 1) % 2
    gcp[lb].wait()
    wcp[lb] = pltpu.async_copy(
        rows_v.at[lb], xg_hbm.at[pl.ds(base + (nk - 1) * GCHUNK, GCHUNK)],
        wsem.at[lb])
    wcp[(nk - 2) % 2].wait()
    wcp[lb].wait()


def _dispatch_call(tok, xf):
    mesh = plsc.VectorSubcoreMesh(core_axis_name="c", subcore_axis_name="s")
    return pl.kernel(
        _dispatch_body,
        out_type=jax.ShapeDtypeStruct((N_SLOT, D_MODEL), jnp.float32),
        mesh=mesh,
        compiler_params=pltpu.CompilerParams(needs_layout_passes=False),
        scratch_types=[
            pltpu.VMEM((SLOTS_PER_W,), jnp.int32),
            pltpu.VMEM((2, GCHUNK, D_MODEL), jnp.float32),
            pltpu.SemaphoreType.DMA((2,)),
            pltpu.SemaphoreType.DMA((2,)),
        ],
    )(tok, xf)


def _ffn_body(cnt_ref, xg_ref, wu_ref, wd_ref, out_ref):
    e = pl.program_id(0)
    xb = xg_ref[...]
    wu = wu_ref[0]
    h = lax.dot_general(xb, wu, (((1,), (1,)), ((), ())),
                        preferred_element_type=jnp.float32)
    u = h[:, :D_FF]
    g = h[:, D_FF:]
    act = u * g * (1.0 / (1.0 + jnp.exp(-g)))
    wd = wd_ref[0]
    y = lax.dot_general(act, wd, (((1,), (1,)), ((), ())),
                        preferred_element_type=jnp.float32)
    cnt = cnt_ref[e]
    rmask = lax.broadcasted_iota(jnp.int32, (CAP, 1), 0) < cnt
    out_ref[...] = jnp.where(rmask, y, 0.0)


def _ffn_call(counts, xg, W_up, W_down):
    grid_spec = pltpu.PrefetchScalarGridSpec(
        num_scalar_prefetch=1,
        grid=(N_EXP,),
        in_specs=[
            pl.BlockSpec((CAP, D_MODEL), lambda e, c: (e, 0)),
            pl.BlockSpec((1, 2 * D_FF, D_MODEL), lambda e, c: (e, 0, 0)),
            pl.BlockSpec((1, D_MODEL, D_FF), lambda e, c: (e, 0, 0)),
        ],
        out_specs=pl.BlockSpec((CAP, D_MODEL), lambda e, c: (e, 0)),
    )
    return pl.pallas_call(
        _ffn_body,
        grid_spec=grid_spec,
        out_shape=jax.ShapeDtypeStruct((N_SLOT, D_MODEL), jnp.float32),
    )(counts, xg, W_up, W_down)


def _combine_body(ye_hbm, slot_hbm, y_hbm, idx_v, rows_v, sem):
    wid = lax.axis_index("s") * 2 + lax.axis_index("c")
    base = wid * TOKS_PER_W
    pltpu.sync_copy(slot_hbm.at[pl.ds(base, TOKS_PER_W)], idx_v)
    pltpu.async_copy(ye_hbm.at[idx_v], rows_v, sem).wait()
    pltpu.sync_copy(rows_v, y_hbm.at[pl.ds(base, TOKS_PER_W)])


def _combine_call(ye, slot):
    mesh = plsc.VectorSubcoreMesh(core_axis_name="c", subcore_axis_name="s")
    return pl.kernel(
        _combine_body,
        out_type=jax.ShapeDtypeStruct((N_TOK, D_MODEL), jnp.float32),
        mesh=mesh,
        compiler_params=pltpu.CompilerParams(needs_layout_passes=False),
        scratch_types=[
            pltpu.VMEM((TOKS_PER_W,), jnp.int32),
            pltpu.VMEM((TOKS_PER_W, D_MODEL), jnp.float32),
            pltpu.SemaphoreType.DMA,
        ],
    )(ye, slot)


@jax.jit
def kernel(x, W_up, W_down, gate_W, gate_b):
    B, T, D = x.shape
    xf = x.reshape(-1, D)
    slot2d, cnt2d, tok2d = _gate_call(xf, gate_W, gate_b)
    slot = slot2d.reshape(-1)
    counts = cnt2d.reshape(-1)
    tok = tok2d.reshape(-1)
    xg = _dispatch_call(tok, xf)
    ye = _ffn_call(counts, xg, W_up, W_down)
    y = _combine_call(ye, slot)
    return y.reshape(B, T, D)


# speedup vs baseline: 1.2324x; 1.0013x over previous
import functools

import jax
import jax.numpy as jnp
from jax import lax
from jax.experimental import pallas as pl
from jax.experimental.pallas import tpu as pltpu
from jax.experimental.pallas import tpu_sc as plsc

D_MODEL = 768
D_FF = 1024
N_EXP = 64
CAP = 160
N_TOK = 2048
N_SLOT = N_EXP * CAP

NW = 32
SLOTS_PER_W = N_SLOT // NW
GCHUNK = 80
TOKS_PER_W = N_TOK // NW


def _gate_body(x_ref, gw_ref, gb_ref, slot_ref, cnt_ref, tok_ref):
    x = x_ref[...]
    gw = gw_ref[...]
    logits = lax.dot_general(x, gw, (((1,), (1,)), ((), ())),
                             preferred_element_type=jnp.float32)
    logits = logits + gb_ref[...]
    m = jnp.max(logits, axis=1, keepdims=True)
    lane = lax.broadcasted_iota(jnp.int32, (N_TOK, N_EXP), 1)
    eid = jnp.min(jnp.where(logits == m, lane, N_EXP), axis=1, keepdims=True)
    onehot = (lane == eid)
    r_i = lax.broadcasted_iota(jnp.int32, (N_TOK, N_TOK), 0)
    c_i = lax.broadcasted_iota(jnp.int32, (N_TOK, N_TOK), 1)
    tri = (c_i < r_i).astype(jnp.bfloat16)
    oh_b = onehot.astype(jnp.bfloat16)
    csum = lax.dot_general(tri, oh_b, (((1,), (0,)), ((), ())),
                           preferred_element_type=jnp.float32)
    rank = jnp.sum(jnp.where(onehot, csum, 0.0), axis=1, keepdims=True)
    rank = rank.astype(jnp.int32)
    counts = jnp.sum(oh_b.astype(jnp.float32), axis=0, keepdims=True)
    cnt_ref[...] = counts.astype(jnp.int32)
    cmin = jnp.min(counts, axis=1, keepdims=True)
    elane = lax.broadcasted_iota(jnp.int32, (1, N_EXP), 1)
    emin = jnp.min(jnp.where(counts == cmin, elane, N_EXP), axis=1,
                   keepdims=True)
    trash = emin * CAP + (CAP - 1)
    kept = rank < CAP
    slot = jnp.where(kept, eid * CAP + rank, trash)
    slot_ref[...] = slot
    tvec = lax.broadcasted_iota(jnp.int32, (N_TOK, 1), 0).astype(jnp.float32)
    ot = onehot.astype(jnp.float32) * tvec
    rmat = (lax.broadcasted_iota(jnp.int32, (N_TOK, CAP), 1) == rank)
    tok = lax.dot_general(ot, rmat.astype(jnp.float32),
                          (((0,), (0,)), ((), ())),
                          preferred_element_type=jnp.float32)
    tok_ref[...] = tok.astype(jnp.int32)


def _gate_call(xf, gate_W, gate_b):
    return pl.pallas_call(
        _gate_body,
        out_shape=[
            jax.ShapeDtypeStruct((N_TOK, 1), jnp.int32),
            jax.ShapeDtypeStruct((1, N_EXP), jnp.int32),
            jax.ShapeDtypeStruct((N_EXP, CAP), jnp.int32),
        ],
    )(xf, gate_W, gate_b.reshape(1, N_EXP))


def _dispatch_body(tok_hbm, x_hbm, xg_hbm, idx0_v, idx1_v, rows_v, gsem, wsem):
    wid = lax.axis_index("s") * 2 + lax.axis_index("c")
    base = wid * SLOTS_PER_W
    idx = [idx0_v, idx1_v]
    nk = SLOTS_PER_W // GCHUNK
    gcp = [None, None]
    wcp = [None, None]
    for k in range(nk):
        b = k % 2
        if wcp[b] is not None:
            wcp[b].wait()
        pltpu.sync_copy(tok_hbm.at[pl.ds(base + k * GCHUNK, GCHUNK)], idx[b])
        gcp[b] = pltpu.async_copy(x_hbm.at[idx[b]], rows_v.at[b], gsem.at[b])
        if k >= 1:
            pb = (k - 1) % 2
            gcp[pb].wait()
            wcp[pb] = pltpu.async_copy(
                rows_v.at[pb], xg_hbm.at[pl.ds(base + (k - 1) * GCHUNK, GCHUNK)],
                wsem.at[pb])
    lb = (nk - 1) % 2
    gcp[lb].wait()
    wcp[lb] = pltpu.async_copy(
        rows_v.at[lb], xg_hbm.at[pl.ds(base + (nk - 1) * GCHUNK, GCHUNK)],
        wsem.at[lb])
    wcp[(nk - 2) % 2].wait()
    wcp[lb].wait()


def _dispatch_call(tok, xf):
    mesh = plsc.VectorSubcoreMesh(core_axis_name="c", subcore_axis_name="s")
    return pl.kernel(
        _dispatch_body,
        out_type=jax.ShapeDtypeStruct((N_SLOT, D_MODEL), jnp.float32),
        mesh=mesh,
        compiler_params=pltpu.CompilerParams(needs_layout_passes=False),
        scratch_types=[
            pltpu.VMEM((GCHUNK,), jnp.int32),
            pltpu.VMEM((GCHUNK,), jnp.int32),
            pltpu.VMEM((2, GCHUNK, D_MODEL), jnp.float32),
            pltpu.SemaphoreType.DMA((2,)),
            pltpu.SemaphoreType.DMA((2,)),
        ],
    )(tok, xf)


def _ffn_body(cnt_ref, xg_ref, wu_ref, wd_ref, out_ref):
    e = pl.program_id(0)
    xb = xg_ref[...]
    wu = wu_ref[0]
    h = lax.dot_general(xb, wu, (((1,), (1,)), ((), ())),
                        preferred_element_type=jnp.float32)
    u = h[:, :D_FF]
    g = h[:, D_FF:]
    act = u * g * (1.0 / (1.0 + jnp.exp(-g)))
    wd = wd_ref[0]
    y = lax.dot_general(act, wd, (((1,), (1,)), ((), ())),
                        preferred_element_type=jnp.float32)
    cnt = cnt_ref[e]
    rmask = lax.broadcasted_iota(jnp.int32, (CAP, 1), 0) < cnt
    out_ref[...] = jnp.where(rmask, y, 0.0)


def _ffn_call(counts, xg, W_up, W_down):
    grid_spec = pltpu.PrefetchScalarGridSpec(
        num_scalar_prefetch=1,
        grid=(N_EXP,),
        in_specs=[
            pl.BlockSpec((CAP, D_MODEL), lambda e, c: (e, 0)),
            pl.BlockSpec((1, 2 * D_FF, D_MODEL), lambda e, c: (e, 0, 0)),
            pl.BlockSpec((1, D_MODEL, D_FF), lambda e, c: (e, 0, 0)),
        ],
        out_specs=pl.BlockSpec((CAP, D_MODEL), lambda e, c: (e, 0)),
    )
    return pl.pallas_call(
        _ffn_body,
        grid_spec=grid_spec,
        out_shape=jax.ShapeDtypeStruct((N_SLOT, D_MODEL), jnp.float32),
    )(counts, xg, W_up, W_down)


def _combine_body(ye_hbm, slot_hbm, y_hbm, idx_v, rows_v, sem):
    wid = lax.axis_index("s") * 2 + lax.axis_index("c")
    base = wid * TOKS_PER_W
    pltpu.sync_copy(slot_hbm.at[pl.ds(base, TOKS_PER_W)], idx_v)
    pltpu.async_copy(ye_hbm.at[idx_v], rows_v, sem).wait()
    pltpu.sync_copy(rows_v, y_hbm.at[pl.ds(base, TOKS_PER_W)])


def _combine_call(ye, slot):
    mesh = plsc.VectorSubcoreMesh(core_axis_name="c", subcore_axis_name="s")
    return pl.kernel(
        _combine_body,
        out_type=jax.ShapeDtypeStruct((N_TOK, D_MODEL), jnp.float32),
        mesh=mesh,
        compiler_params=pltpu.CompilerParams(needs_layout_passes=False),
        scratch_types=[
            pltpu.VMEM((TOKS_PER_W,), jnp.int32),
            pltpu.VMEM((TOKS_PER_W, D_MODEL), jnp.float32),
            pltpu.SemaphoreType.DMA,
        ],
    )(ye, slot)


@jax.jit
def kernel(x, W_up, W_down, gate_W, gate_b):
    B, T, D = x.shape
    xf = x.reshape(-1, D)
    slot2d, cnt2d, tok2d = _gate_call(xf, gate_W, gate_b)
    slot = slot2d.reshape(-1)
    counts = cnt2d.reshape(-1)
    tok = tok2d.reshape(-1)
    xg = _dispatch_call(tok, xf)
    ye = _ffn_call(counts, xg, W_up, W_down)
    y = _combine_call(ye, slot)
    return y.reshape(B, T, D)


# speedup vs baseline: 3.0709x; 2.4918x over previous
import functools

import jax
import jax.numpy as jnp
from jax import lax
from jax.experimental import pallas as pl
from jax.experimental.pallas import tpu as pltpu
from jax.experimental.pallas import tpu_sc as plsc

D_MODEL = 768
D_FF = 1024
N_EXP = 64
CAP = 160
N_TOK = 2048
N_SLOT = N_EXP * CAP

NW = 32
TOKS_PER_W = N_TOK // NW


def _gate_body(x_ref, gw_ref, gb_ref, slot_ref, cnt_ref):
    x = x_ref[...]
    gw = gw_ref[...]
    logits = lax.dot_general(x, gw, (((1,), (1,)), ((), ())),
                             preferred_element_type=jnp.float32)
    logits = logits + gb_ref[...]
    m = jnp.max(logits, axis=1, keepdims=True)
    lane = lax.broadcasted_iota(jnp.int32, (N_TOK, N_EXP), 1)
    eid = jnp.min(jnp.where(logits == m, lane, N_EXP), axis=1, keepdims=True)
    onehot = (lane == eid)
    r_i = lax.broadcasted_iota(jnp.int32, (N_TOK, N_TOK), 0)
    c_i = lax.broadcasted_iota(jnp.int32, (N_TOK, N_TOK), 1)
    tri = (c_i < r_i).astype(jnp.bfloat16)
    oh_b = onehot.astype(jnp.bfloat16)
    csum = lax.dot_general(tri, oh_b, (((1,), (0,)), ((), ())),
                           preferred_element_type=jnp.float32)
    rank = jnp.sum(jnp.where(onehot, csum, 0.0), axis=1, keepdims=True)
    rank = rank.astype(jnp.int32)
    counts = jnp.sum(oh_b.astype(jnp.float32), axis=0, keepdims=True)
    cnt_ref[...] = counts.astype(jnp.int32)
    cmin = jnp.min(counts, axis=1, keepdims=True)
    elane = lax.broadcasted_iota(jnp.int32, (1, N_EXP), 1)
    emin = jnp.min(jnp.where(counts == cmin, elane, N_EXP), axis=1,
                   keepdims=True)
    trash = emin * CAP + (CAP - 1)
    kept = rank < CAP
    slot = jnp.where(kept, eid * CAP + rank, trash)
    slot_ref[...] = slot


def _gate_call(xf, gate_W, gate_b):
    return pl.pallas_call(
        _gate_body,
        out_shape=[
            jax.ShapeDtypeStruct((N_TOK, 1), jnp.int32),
            jax.ShapeDtypeStruct((1, N_EXP), jnp.int32),
        ],
    )(xf, gate_W, gate_b.reshape(1, N_EXP))


def _dispatch_body(slot_hbm, x_hbm, xg_hbm, idx_v, rows_v, sem):
    wid = lax.axis_index("s") * 2 + lax.axis_index("c")
    base = wid * TOKS_PER_W
    pltpu.sync_copy(slot_hbm.at[pl.ds(base, TOKS_PER_W)], idx_v)
    pltpu.sync_copy(x_hbm.at[pl.ds(base, TOKS_PER_W)], rows_v)
    pltpu.async_copy(rows_v, xg_hbm.at[idx_v], sem).wait()


def _dispatch_call(slot, xf):
    mesh = plsc.VectorSubcoreMesh(core_axis_name="c", subcore_axis_name="s")
    return pl.kernel(
        _dispatch_body,
        out_type=jax.ShapeDtypeStruct((N_SLOT, D_MODEL), jnp.float32),
        mesh=mesh,
        compiler_params=pltpu.CompilerParams(needs_layout_passes=False),
        scratch_types=[
            pltpu.VMEM((TOKS_PER_W,), jnp.int32),
            pltpu.VMEM((TOKS_PER_W, D_MODEL), jnp.float32),
            pltpu.SemaphoreType.DMA,
        ],
    )(slot, xf)


def _ffn_body(cnt_ref, xg_ref, wu_ref, wd_ref, out_ref):
    e = pl.program_id(0)
    xb = xg_ref[...]
    wu = wu_ref[0]
    h = lax.dot_general(xb, wu, (((1,), (1,)), ((), ())),
                        preferred_element_type=jnp.float32)
    u = h[:, :D_FF]
    g = h[:, D_FF:]
    act = u * g * (1.0 / (1.0 + jnp.exp(-g)))
    wd = wd_ref[0]
    y = lax.dot_general(act, wd, (((1,), (1,)), ((), ())),
                        preferred_element_type=jnp.float32)
    cnt = cnt_ref[e]
    rmask = lax.broadcasted_iota(jnp.int32, (CAP, 1), 0) < cnt
    out_ref[...] = jnp.where(rmask, y, 0.0)


def _ffn_call(counts, xg, W_up, W_down):
    grid_spec = pltpu.PrefetchScalarGridSpec(
        num_scalar_prefetch=1,
        grid=(N_EXP,),
        in_specs=[
            pl.BlockSpec((CAP, D_MODEL), lambda e, c: (e, 0)),
            pl.BlockSpec((1, 2 * D_FF, D_MODEL), lambda e, c: (e, 0, 0)),
            pl.BlockSpec((1, D_MODEL, D_FF), lambda e, c: (e, 0, 0)),
        ],
        out_specs=pl.BlockSpec((CAP, D_MODEL), lambda e, c: (e, 0)),
    )
    return pl.pallas_call(
        _ffn_body,
        grid_spec=grid_spec,
        out_shape=jax.ShapeDtypeStruct((N_SLOT, D_MODEL), jnp.float32),
    )(counts, xg, W_up, W_down)


def _combine_body(ye_hbm, slot_hbm, y_hbm, idx_v, rows_v, sem):
    wid = lax.axis_index("s") * 2 + lax.axis_index("c")
    base = wid * TOKS_PER_W
    pltpu.sync_copy(slot_hbm.at[pl.ds(base, TOKS_PER_W)], idx_v)
    pltpu.async_copy(ye_hbm.at[idx_v], rows_v, sem).wait()
    pltpu.sync_copy(rows_v, y_hbm.at[pl.ds(base, TOKS_PER_W)])


def _combine_call(ye, slot):
    mesh = plsc.VectorSubcoreMesh(core_axis_name="c", subcore_axis_name="s")
    return pl.kernel(
        _combine_body,
        out_type=jax.ShapeDtypeStruct((N_TOK, D_MODEL), jnp.float32),
        mesh=mesh,
        compiler_params=pltpu.CompilerParams(needs_layout_passes=False),
        scratch_types=[
            pltpu.VMEM((TOKS_PER_W,), jnp.int32),
            pltpu.VMEM((TOKS_PER_W, D_MODEL), jnp.float32),
            pltpu.SemaphoreType.DMA,
        ],
    )(ye, slot)


@jax.jit
def kernel(x, W_up, W_down, gate_W, gate_b):
    B, T, D = x.shape
    xf = x.reshape(-1, D)
    slot2d, cnt2d = _gate_call(xf, gate_W, gate_b)
    slot = slot2d.reshape(-1)
    counts = cnt2d.reshape(-1)
    xg = _dispatch_call(slot, xf)
    ye = _ffn_call(counts, xg, W_up, W_down)
    y = _combine_call(ye, slot)
    return y.reshape(B, T, D)


# speedup vs baseline: 3.1074x; 1.0119x over previous
import functools

import jax
import jax.numpy as jnp
from jax import lax
from jax.experimental import pallas as pl
from jax.experimental.pallas import tpu as pltpu
from jax.experimental.pallas import tpu_sc as plsc

D_MODEL = 768
D_FF = 1024
N_EXP = 64
CAP = 160
N_TOK = 2048
N_SLOT = N_EXP * CAP

NW = 32
TOKS_PER_W = N_TOK // NW


def _gate_body(x_ref, gw_ref, gb_ref, slot_ref, cnt_ref):
    x = x_ref[...]
    gw = gw_ref[...]
    logits = lax.dot_general(x, gw, (((1,), (1,)), ((), ())),
                             preferred_element_type=jnp.float32)
    logits = logits + gb_ref[...]
    m = jnp.max(logits, axis=1, keepdims=True)
    lane = lax.broadcasted_iota(jnp.int32, (N_TOK, N_EXP), 1)
    eid = jnp.min(jnp.where(logits == m, lane, N_EXP), axis=1, keepdims=True)
    onehot = (lane == eid)
    oh_b = onehot.astype(jnp.bfloat16)
    CH = 256
    r_i = lax.broadcasted_iota(jnp.int32, (CH, CH), 0)
    c_i = lax.broadcasted_iota(jnp.int32, (CH, CH), 1)
    tri = (c_i < r_i).astype(jnp.bfloat16)
    csum_parts = []
    carry = jnp.zeros((1, N_EXP), jnp.float32)
    for j in range(N_TOK // CH):
        blk = oh_b[j * CH:(j + 1) * CH]
        local = lax.dot_general(tri, blk, (((1,), (0,)), ((), ())),
                                preferred_element_type=jnp.float32)
        csum_parts.append(local + carry)
        carry = carry + jnp.sum(blk.astype(jnp.float32), axis=0, keepdims=True)
    csum = jnp.concatenate(csum_parts, axis=0)
    rank = jnp.sum(jnp.where(onehot, csum, 0.0), axis=1, keepdims=True)
    rank = rank.astype(jnp.int32)
    counts = carry
    cnt_ref[...] = counts.astype(jnp.int32)
    cmin = jnp.min(counts, axis=1, keepdims=True)
    elane = lax.broadcasted_iota(jnp.int32, (1, N_EXP), 1)
    emin = jnp.min(jnp.where(counts == cmin, elane, N_EXP), axis=1,
                   keepdims=True)
    trash = emin * CAP + (CAP - 1)
    kept = rank < CAP
    slot = jnp.where(kept, eid * CAP + rank, trash)
    slot_ref[...] = slot


def _gate_call(xf, gate_W, gate_b):
    return pl.pallas_call(
        _gate_body,
        out_shape=[
            jax.ShapeDtypeStruct((N_TOK, 1), jnp.int32),
            jax.ShapeDtypeStruct((1, N_EXP), jnp.int32),
        ],
    )(xf, gate_W, gate_b.reshape(1, N_EXP))


def _dispatch_body(slot_hbm, x_hbm, xg_hbm, idx_v, rows_v, sem):
    wid = lax.axis_index("s") * 2 + lax.axis_index("c")
    base = wid * TOKS_PER_W
    pltpu.sync_copy(slot_hbm.at[pl.ds(base, TOKS_PER_W)], idx_v)
    pltpu.sync_copy(x_hbm.at[pl.ds(base, TOKS_PER_W)], rows_v)
    pltpu.async_copy(rows_v, xg_hbm.at[idx_v], sem).wait()


def _dispatch_call(slot, xf):
    mesh = plsc.VectorSubcoreMesh(core_axis_name="c", subcore_axis_name="s")
    return pl.kernel(
        _dispatch_body,
        out_type=jax.ShapeDtypeStruct((N_SLOT, D_MODEL), jnp.float32),
        mesh=mesh,
        compiler_params=pltpu.CompilerParams(needs_layout_passes=False),
        scratch_types=[
            pltpu.VMEM((TOKS_PER_W,), jnp.int32),
            pltpu.VMEM((TOKS_PER_W, D_MODEL), jnp.float32),
            pltpu.SemaphoreType.DMA,
        ],
    )(slot, xf)


def _ffn_body(cnt_ref, xg_ref, wu_ref, wd_ref, out_ref):
    e = pl.program_id(0)
    xb = xg_ref[...]
    wu = wu_ref[0]
    h = lax.dot_general(xb, wu, (((1,), (1,)), ((), ())),
                        preferred_element_type=jnp.float32)
    u = h[:, :D_FF]
    g = h[:, D_FF:]
    act = u * g * (1.0 / (1.0 + jnp.exp(-g)))
    wd = wd_ref[0]
    y = lax.dot_general(act, wd, (((1,), (1,)), ((), ())),
                        preferred_element_type=jnp.float32)
    cnt = cnt_ref[e]
    rmask = lax.broadcasted_iota(jnp.int32, (CAP, 1), 0) < cnt
    out_ref[...] = jnp.where(rmask, y, 0.0)


def _ffn_call(counts, xg, W_up, W_down):
    grid_spec = pltpu.PrefetchScalarGridSpec(
        num_scalar_prefetch=1,
        grid=(N_EXP,),
        in_specs=[
            pl.BlockSpec((CAP, D_MODEL), lambda e, c: (e, 0)),
            pl.BlockSpec((1, 2 * D_FF, D_MODEL), lambda e, c: (e, 0, 0)),
            pl.BlockSpec((1, D_MODEL, D_FF), lambda e, c: (e, 0, 0)),
        ],
        out_specs=pl.BlockSpec((CAP, D_MODEL), lambda e, c: (e, 0)),
    )
    return pl.pallas_call(
        _ffn_body,
        grid_spec=grid_spec,
        out_shape=jax.ShapeDtypeStruct((N_SLOT, D_MODEL), jnp.float32),
    )(counts, xg, W_up, W_down)


def _combine_body(ye_hbm, slot_hbm, y_hbm, idx_v, rows_v, sem):
    wid = lax.axis_index("s") * 2 + lax.axis_index("c")
    base = wid * TOKS_PER_W
    pltpu.sync_copy(slot_hbm.at[pl.ds(base, TOKS_PER_W)], idx_v)
    pltpu.async_copy(ye_hbm.at[idx_v], rows_v, sem).wait()
    pltpu.sync_copy(rows_v, y_hbm.at[pl.ds(base, TOKS_PER_W)])


def _combine_call(ye, slot):
    mesh = plsc.VectorSubcoreMesh(core_axis_name="c", subcore_axis_name="s")
    return pl.kernel(
        _combine_body,
        out_type=jax.ShapeDtypeStruct((N_TOK, D_MODEL), jnp.float32),
        mesh=mesh,
        compiler_params=pltpu.CompilerParams(needs_layout_passes=False),
        scratch_types=[
            pltpu.VMEM((TOKS_PER_W,), jnp.int32),
            pltpu.VMEM((TOKS_PER_W, D_MODEL), jnp.float32),
            pltpu.SemaphoreType.DMA,
        ],
    )(ye, slot)


@jax.jit
def kernel(x, W_up, W_down, gate_W, gate_b):
    B, T, D = x.shape
    xf = x.reshape(-1, D)
    slot2d, cnt2d = _gate_call(xf, gate_W, gate_b)
    slot = slot2d.reshape(-1)
    counts = cnt2d.reshape(-1)
    xg = _dispatch_call(slot, xf)
    ye = _ffn_call(counts, xg, W_up, W_down)
    y = _combine_call(ye, slot)
    return y.reshape(B, T, D)


# speedup vs baseline: 3.1227x; 1.0049x over previous
import functools

import jax
import jax.numpy as jnp
from jax import lax
from jax.experimental import pallas as pl
from jax.experimental.pallas import tpu as pltpu
from jax.experimental.pallas import tpu_sc as plsc

D_MODEL = 768
D_FF = 1024
N_EXP = 64
CAP = 160
N_TOK = 2048
N_SLOT = N_EXP * CAP

NW = 32
TOKS_PER_W = N_TOK // NW


def _gate_body(x_ref, gw_ref, gb_ref, slot_ref, cnt_ref):
    x = x_ref[...]
    gw = gw_ref[...]
    logits = lax.dot_general(x, gw, (((1,), (1,)), ((), ())),
                             preferred_element_type=jnp.float32)
    logits = logits + gb_ref[...]
    m = jnp.max(logits, axis=1, keepdims=True)
    lane = lax.broadcasted_iota(jnp.int32, (N_TOK, N_EXP), 1)
    eid = jnp.min(jnp.where(logits == m, lane, N_EXP), axis=1, keepdims=True)
    onehot = (lane == eid)
    oh_b = onehot.astype(jnp.bfloat16)
    CH = 256
    r_i = lax.broadcasted_iota(jnp.int32, (CH, CH), 0)
    c_i = lax.broadcasted_iota(jnp.int32, (CH, CH), 1)
    tri = (c_i < r_i).astype(jnp.bfloat16)
    csum_parts = []
    carry = jnp.zeros((1, N_EXP), jnp.float32)
    for j in range(N_TOK // CH):
        blk = oh_b[j * CH:(j + 1) * CH]
        local = lax.dot_general(tri, blk, (((1,), (0,)), ((), ())),
                                preferred_element_type=jnp.float32)
        csum_parts.append(local + carry)
        carry = carry + jnp.sum(blk.astype(jnp.float32), axis=0, keepdims=True)
    csum = jnp.concatenate(csum_parts, axis=0)
    rank = jnp.sum(jnp.where(onehot, csum, 0.0), axis=1, keepdims=True)
    rank = rank.astype(jnp.int32)
    counts = carry
    cnt_ref[...] = counts.astype(jnp.int32)
    cmin = jnp.min(counts, axis=1, keepdims=True)
    elane = lax.broadcasted_iota(jnp.int32, (1, N_EXP), 1)
    emin = jnp.min(jnp.where(counts == cmin, elane, N_EXP), axis=1,
                   keepdims=True)
    trash = emin * CAP + (CAP - 1)
    kept = rank < CAP
    slot = jnp.where(kept, eid * CAP + rank, trash)
    slot_ref[...] = slot.reshape(N_TOK)


def _gate_call(xf, gate_W, gate_b):
    return pl.pallas_call(
        _gate_body,
        out_shape=[
            jax.ShapeDtypeStruct((N_TOK,), jnp.int32),
            jax.ShapeDtypeStruct((1, N_EXP), jnp.int32),
        ],
    )(xf, gate_W, gate_b.reshape(1, N_EXP))


def _dispatch_body(slot_hbm, x_hbm, xg_hbm, idx0_v, idx1_v, rows0_v, rows1_v,
                   sem):
    wid = lax.axis_index("s") * 2 + lax.axis_index("c")
    base = wid * TOKS_PER_W
    H = TOKS_PER_W // 2
    idxs = [idx0_v, idx1_v]
    rows = [rows0_v, rows1_v]
    cp = [None, None]
    for k in range(2):
        pltpu.sync_copy(slot_hbm.at[pl.ds(base + k * H, H)], idxs[k])
        pltpu.sync_copy(x_hbm.at[pl.ds(base + k * H, H)], rows[k])
        cp[k] = pltpu.async_copy(rows[k], xg_hbm.at[idxs[k]], sem.at[k])
    cp[0].wait()
    cp[1].wait()


def _dispatch_call(slot, xf):
    mesh = plsc.VectorSubcoreMesh(core_axis_name="c", subcore_axis_name="s")
    return pl.kernel(
        _dispatch_body,
        out_type=jax.ShapeDtypeStruct((N_SLOT, D_MODEL), jnp.float32),
        mesh=mesh,
        compiler_params=pltpu.CompilerParams(needs_layout_passes=False),
        scratch_types=[
            pltpu.VMEM((TOKS_PER_W // 2,), jnp.int32),
            pltpu.VMEM((TOKS_PER_W // 2,), jnp.int32),
            pltpu.VMEM((TOKS_PER_W // 2, D_MODEL), jnp.float32),
            pltpu.VMEM((TOKS_PER_W // 2, D_MODEL), jnp.float32),
            pltpu.SemaphoreType.DMA((2,)),
        ],
    )(slot, xf)


def _ffn_body(cnt_ref, xg_ref, wu_ref, wd_ref, out_ref):
    e = pl.program_id(0)
    xb = xg_ref[...]
    wu = wu_ref[0]
    h = lax.dot_general(xb, wu, (((1,), (1,)), ((), ())),
                        preferred_element_type=jnp.float32)
    u = h[:, :D_FF]
    g = h[:, D_FF:]
    act = u * g * (1.0 / (1.0 + jnp.exp(-g)))
    wd = wd_ref[0]
    y = lax.dot_general(act, wd, (((1,), (1,)), ((), ())),
                        preferred_element_type=jnp.float32)
    cnt = cnt_ref[e]
    rmask = lax.broadcasted_iota(jnp.int32, (CAP, 1), 0) < cnt
    out_ref[...] = jnp.where(rmask, y, 0.0)


def _ffn_call(counts, xg, W_up, W_down):
    grid_spec = pltpu.PrefetchScalarGridSpec(
        num_scalar_prefetch=1,
        grid=(N_EXP,),
        in_specs=[
            pl.BlockSpec((CAP, D_MODEL), lambda e, c: (e, 0)),
            pl.BlockSpec((1, 2 * D_FF, D_MODEL), lambda e, c: (e, 0, 0)),
            pl.BlockSpec((1, D_MODEL, D_FF), lambda e, c: (e, 0, 0)),
        ],
        out_specs=pl.BlockSpec((CAP, D_MODEL), lambda e, c: (e, 0)),
    )
    return pl.pallas_call(
        _ffn_body,
        grid_spec=grid_spec,
        out_shape=jax.ShapeDtypeStruct((N_SLOT, D_MODEL), jnp.float32),
    )(counts, xg, W_up, W_down)


def _combine_body(ye_hbm, slot_hbm, y_hbm, idx0_v, idx1_v, rows0_v, rows1_v,
                  gsem, wsem):
    wid = lax.axis_index("s") * 2 + lax.axis_index("c")
    base = wid * TOKS_PER_W
    H = TOKS_PER_W // 2
    idxs = [idx0_v, idx1_v]
    rows = [rows0_v, rows1_v]
    gcp = [None, None]
    wcp = [None, None]
    for k in range(2):
        pltpu.sync_copy(slot_hbm.at[pl.ds(base + k * H, H)], idxs[k])
        gcp[k] = pltpu.async_copy(ye_hbm.at[idxs[k]], rows[k], gsem.at[k])
    for k in range(2):
        gcp[k].wait()
        wcp[k] = pltpu.async_copy(rows[k], y_hbm.at[pl.ds(base + k * H, H)],
                                  wsem.at[k])
    wcp[0].wait()
    wcp[1].wait()


def _combine_call(ye, slot):
    mesh = plsc.VectorSubcoreMesh(core_axis_name="c", subcore_axis_name="s")
    return pl.kernel(
        _combine_body,
        out_type=jax.ShapeDtypeStruct((N_TOK, D_MODEL), jnp.float32),
        mesh=mesh,
        compiler_params=pltpu.CompilerParams(needs_layout_passes=False),
        scratch_types=[
            pltpu.VMEM((TOKS_PER_W // 2,), jnp.int32),
            pltpu.VMEM((TOKS_PER_W // 2,), jnp.int32),
            pltpu.VMEM((TOKS_PER_W // 2, D_MODEL), jnp.float32),
            pltpu.VMEM((TOKS_PER_W // 2, D_MODEL), jnp.float32),
            pltpu.SemaphoreType.DMA((2,)),
            pltpu.SemaphoreType.DMA((2,)),
        ],
    )(ye, slot)


@jax.jit
def kernel(x, W_up, W_down, gate_W, gate_b):
    B, T, D = x.shape
    xf = x.reshape(-1, D)
    slot, cnt2d = _gate_call(xf, gate_W, gate_b)
    counts = cnt2d.reshape(-1)
    xg = _dispatch_call(slot, xf)
    ye = _ffn_call(counts, xg, W_up, W_down)
    y = _combine_call(ye, slot)
    return y.reshape(B, T, D)


# speedup vs baseline: 3.1340x; 1.0036x over previous
import jax
import jax.numpy as jnp
from jax import lax
from jax.experimental import pallas as pl
from jax.experimental.pallas import tpu as pltpu
from jax.experimental.pallas import tpu_sc as plsc

D_MODEL = 768
D_FF = 1024
N_EXP = 64
CAP = 160
N_TOK = 2048
N_SLOT = N_EXP * CAP

NW = 32
TOKS_PER_W = N_TOK // NW


def _gate_body(x_ref, gw_ref, gb_ref, slot_ref, cnt_ref):
    x = x_ref[...]
    gw = gw_ref[...]
    logits = lax.dot_general(x, gw, (((1,), (1,)), ((), ())),
                             preferred_element_type=jnp.float32)
    logits = logits + gb_ref[...]
    m = jnp.max(logits, axis=1, keepdims=True)
    lane = lax.broadcasted_iota(jnp.int32, (N_TOK, N_EXP), 1)
    eid = jnp.min(jnp.where(logits == m, lane, N_EXP), axis=1, keepdims=True)
    onehot = (lane == eid)
    oh_b = onehot.astype(jnp.bfloat16)
    CH = 256
    r_i = lax.broadcasted_iota(jnp.int32, (CH, CH), 0)
    c_i = lax.broadcasted_iota(jnp.int32, (CH, CH), 1)
    tri = (c_i < r_i).astype(jnp.bfloat16)
    csum_parts = []
    carry = jnp.zeros((1, N_EXP), jnp.float32)
    for j in range(N_TOK // CH):
        blk = oh_b[j * CH:(j + 1) * CH]
        local = lax.dot_general(tri, blk, (((1,), (0,)), ((), ())),
                                preferred_element_type=jnp.float32)
        csum_parts.append(local + carry)
        carry = carry + jnp.sum(blk.astype(jnp.float32), axis=0, keepdims=True)
    csum = jnp.concatenate(csum_parts, axis=0)
    rank = jnp.sum(jnp.where(onehot, csum, 0.0), axis=1, keepdims=True)
    rank = rank.astype(jnp.int32)
    counts = carry
    cnt_ref[...] = counts.astype(jnp.int32)
    cmin = jnp.min(counts, axis=1, keepdims=True)
    elane = lax.broadcasted_iota(jnp.int32, (1, N_EXP), 1)
    emin = jnp.min(jnp.where(counts == cmin, elane, N_EXP), axis=1,
                   keepdims=True)
    trash = emin * CAP + (CAP - 1)
    kept = rank < CAP
    slot = jnp.where(kept, eid * CAP + rank, trash)
    slot_ref[...] = slot.reshape(N_TOK)


def _gate_call(xf, gate_W, gate_b):
    return pl.pallas_call(
        _gate_body,
        out_shape=[
            jax.ShapeDtypeStruct((N_TOK,), jnp.int32),
            jax.ShapeDtypeStruct((1, N_EXP), jnp.int32),
        ],
    )(xf, gate_W, gate_b.reshape(1, N_EXP))


def _dispatch_body(slot_hbm, x_hbm, xg_hbm, idx0_v, rows0_v, sem):
    wid = lax.axis_index("s") * 2 + lax.axis_index("c")
    base = wid * TOKS_PER_W
    pltpu.sync_copy(slot_hbm.at[pl.ds(base, TOKS_PER_W)], idx0_v)
    pltpu.sync_copy(x_hbm.at[pl.ds(base, TOKS_PER_W)], rows0_v)
    pltpu.async_copy(rows0_v, xg_hbm.at[idx0_v], sem).wait()


def _dispatch_call(slot, xf):
    mesh = plsc.VectorSubcoreMesh(core_axis_name="c", subcore_axis_name="s")
    return pl.kernel(
        _dispatch_body,
        out_type=jax.ShapeDtypeStruct((N_SLOT, D_MODEL), jnp.float32),
        mesh=mesh,
        compiler_params=pltpu.CompilerParams(needs_layout_passes=False),
        scratch_types=[
            pltpu.VMEM((TOKS_PER_W,), jnp.int32),
            pltpu.VMEM((TOKS_PER_W, D_MODEL), jnp.float32),
            pltpu.SemaphoreType.DMA,
        ],
    )(slot, xf)


def _ffn_body(cnt_ref, xg_ref, wu_ref, wd_ref, out_ref):
    e = pl.program_id(0)
    xb = xg_ref[...]
    wu = wu_ref[0]
    h = lax.dot_general(xb, wu, (((1,), (1,)), ((), ())),
                        preferred_element_type=jnp.float32)
    u = h[:, :D_FF]
    g = h[:, D_FF:]
    act = u * g * (1.0 / (1.0 + jnp.exp(-g)))
    wd = wd_ref[0]
    y = lax.dot_general(act, wd, (((1,), (1,)), ((), ())),
                        preferred_element_type=jnp.float32)
    cnt = cnt_ref[e]
    rmask = lax.broadcasted_iota(jnp.int32, (CAP, 1), 0) < cnt
    out_ref[...] = jnp.where(rmask, y, 0.0)


def _ffn_call(counts, xg, W_up, W_down):
    grid_spec = pltpu.PrefetchScalarGridSpec(
        num_scalar_prefetch=1,
        grid=(N_EXP,),
        in_specs=[
            pl.BlockSpec((CAP, D_MODEL), lambda e, c: (e, 0)),
            pl.BlockSpec((1, 2 * D_FF, D_MODEL), lambda e, c: (e, 0, 0)),
            pl.BlockSpec((1, D_MODEL, D_FF), lambda e, c: (e, 0, 0)),
        ],
        out_specs=pl.BlockSpec((CAP, D_MODEL), lambda e, c: (e, 0)),
    )
    return pl.pallas_call(
        _ffn_body,
        grid_spec=grid_spec,
        out_shape=jax.ShapeDtypeStruct((N_SLOT, D_MODEL), jnp.float32),
    )(counts, xg, W_up, W_down)


def _combine_body(ye_hbm, slot_hbm, y_hbm, idx0_v, idx1_v, rows0_v, rows1_v,
                  gsem, wsem):
    wid = lax.axis_index("s") * 2 + lax.axis_index("c")
    base = wid * TOKS_PER_W
    H = TOKS_PER_W // 2
    idxs = [idx0_v, idx1_v]
    rows = [rows0_v, rows1_v]
    gcp = [None, None]
    wcp = [None, None]
    for k in range(2):
        pltpu.sync_copy(slot_hbm.at[pl.ds(base + k * H, H)], idxs[k])
        gcp[k] = pltpu.async_copy(ye_hbm.at[idxs[k]], rows[k], gsem.at[k])
    for k in range(2):
        gcp[k].wait()
        wcp[k] = pltpu.async_copy(rows[k], y_hbm.at[pl.ds(base + k * H, H)],
                                  wsem.at[k])
    wcp[0].wait()
    wcp[1].wait()


def _combine_call(ye, slot):
    mesh = plsc.VectorSubcoreMesh(core_axis_name="c", subcore_axis_name="s")
    return pl.kernel(
        _combine_body,
        out_type=jax.ShapeDtypeStruct((N_TOK, D_MODEL), jnp.float32),
        mesh=mesh,
        compiler_params=pltpu.CompilerParams(needs_layout_passes=False),
        scratch_types=[
            pltpu.VMEM((TOKS_PER_W // 2,), jnp.int32),
            pltpu.VMEM((TOKS_PER_W // 2,), jnp.int32),
            pltpu.VMEM((TOKS_PER_W // 2, D_MODEL), jnp.float32),
            pltpu.VMEM((TOKS_PER_W // 2, D_MODEL), jnp.float32),
            pltpu.SemaphoreType.DMA((2,)),
            pltpu.SemaphoreType.DMA((2,)),
        ],
    )(ye, slot)


@jax.jit
def kernel(x, W_up, W_down, gate_W, gate_b):
    B, T, D = x.shape
    xf = x.reshape(-1, D)
    slot, cnt2d = _gate_call(xf, gate_W, gate_b)
    counts = cnt2d.reshape(-1)
    xg = _dispatch_call(slot, xf)
    ye = _ffn_call(counts, xg, W_up, W_down)
    y = _combine_call(ye, slot)
    return y.reshape(B, T, D)
